# Initial kernel scaffold; baseline (speedup 1.0000x reference)
import jax
import jax.numpy as jnp
from jax.experimental import pallas as pl


def kernel(a1, a2, mb_a1, mb_a2, ft):
    raise NotImplementedError("write your pallas kernel here")


# speedup vs baseline: 2.0555x; 2.0555x over previous
import functools

import jax
import jax.numpy as jnp
from jax.experimental import pallas as pl
from jax.experimental.pallas import tpu as pltpu

BN = 40


def _gat_block(a1_ref, a2_ref, mb_a1_ref, mb_a2_ref, ft_ref, out_ref):
    a1 = a1_ref[...]
    a2 = a2_ref[...]
    mb_a1 = mb_a1_ref[...]
    mb_a2 = mb_a2_ref[...]
    ft = ft_ref[...]

    b = a2[:, None, :] + mb_a1
    s = jnp.sum(a1[:, None, :] + mb_a2 + b, axis=-1, keepdims=True)
    z = s + b
    z = jnp.where(z >= 0, z, 0.01 * z)
    m = jnp.max(z, axis=1, keepdims=True)
    e = jnp.exp(z - m)
    denom = jnp.sum(e, axis=1)
    num = jnp.sum(e * ft_ref[...], axis=1)
    del ft
    out_ref[...] = num / denom


def kernel(a1, a2, mb_a1, mb_a2, ft):
    n, d = a1.shape
    k = mb_a1.shape[1]
    bn = BN
    grid = (n // bn,)
    out = pl.pallas_call(
        _gat_block,
        grid=grid,
        in_specs=[
            pl.BlockSpec((bn, d), lambda i: (i, 0)),
            pl.BlockSpec((bn, d), lambda i: (i, 0)),
            pl.BlockSpec((bn, k, d), lambda i: (i, 0, 0)),
            pl.BlockSpec((bn, k, d), lambda i: (i, 0, 0)),
            pl.BlockSpec((bn, k, d), lambda i: (i, 0, 0)),
        ],
        out_specs=pl.BlockSpec((bn, d), lambda i: (i, 0)),
        out_shape=jax.ShapeDtypeStruct((n, d), jnp.float32),
    )(a1, a2, mb_a1, mb_a2, ft)
    return out
